# baseline (speedup 1.0000x reference)
import functools

import jax
import jax.numpy as jnp
import numpy as np
from jax import lax
from jax.experimental import pallas as pl
from jax.experimental.pallas import tpu as pltpu
from jax.experimental.pallas import tpu_sc as plsc

B = 128
N = 100000
NC, NS, L = 2, 16, 16
NW = NC * NS
RPW = B // NW
CH = 4000
NCH = N // CH
U = 5
BIG = np.float32(3.0e38)


def _xlane(redbuf, lanes, v, op):
    cur = v
    for s in (8, 4, 2, 1):
        redbuf[pl.ds(0, L)] = cur
        idx = jnp.bitwise_and(lanes + s, L - 1)
        cur = op(cur, plsc.load_gather(redbuf, [idx]))
    return cur


def _sc_body(scores_hbm, avail_hbm, action_hbm, probs_hbm, aux_hbm,
             smrow, sbuf, abuf, pbuf, actbuf, auxbuf, redbuf):
    w = lax.axis_index("s") * NC + lax.axis_index("c")
    pltpu.sync_copy(action_hbm, actbuf)
    lanes = lax.iota(jnp.int32, L)
    zeros16 = jnp.zeros((L,), jnp.float32)
    zerosi = jnp.zeros((L,), jnp.int32)

    def row_body(k, _):
        b = w * RPW + k
        base = b * N

        def make_p1(cbase):
            def p1(j, c):
                vm, vn = c
                for u in range(U):
                    off = j * (L * U) + u * L
                    s = sbuf[pl.ds(off, L)]
                    a = abuf[pl.ds(off, L)]
                    msk = a > 0
                    smv = jnp.where(msk, s, -BIG)
                    smrow[pl.ds(cbase + off, L)] = smv
                    vm = jnp.maximum(vm, smv)
                    vn = vn + jnp.where(msk, np.float32(1.0), np.float32(0.0))
                return vm, vn
            return p1

        carry = (jnp.full((L,), -BIG, jnp.float32), zeros16)
        for i in range(NCH):
            pltpu.sync_copy(scores_hbm.at[pl.ds(base + i * CH, CH)], sbuf)
            pltpu.sync_copy(avail_hbm.at[pl.ds(base + i * CH, CH)], abuf)
            carry = lax.fori_loop(0, CH // (L * U), make_p1(i * CH), carry)
        vmax, vnav = carry
        Mf = _xlane(redbuf, lanes, vmax, jnp.maximum)
        nav = _xlane(redbuf, lanes, vnav, jnp.add)

        def p2(j, c):
            vE, vEZ = c
            for u in range(U):
                off = j * (L * U) + u * L
                smv = smrow[pl.ds(off, L)]
                zz = smv - Mf
                ee = jnp.exp(zz)
                smrow[pl.ds(off, L)] = ee
                vE = vE + ee
                vEZ = vEZ + ee * zz
            return vE, vEZ

        vE, vEZ = lax.fori_loop(0, N // (L * U), p2, (zeros16, zeros16))
        E = _xlane(redbuf, lanes, vE, jnp.add)
        EZ = _xlane(redbuf, lanes, vEZ, jnp.add)
        vdenom = E + (np.float32(N) - nav)
        vS = E / vdenom
        vscale = (zeros16 + np.float32(1.0)) / (vdenom * (vS + np.float32(1e-13)))
        vscale = jnp.where(nav > np.float32(0.0), vscale, np.float32(0.0))

        def make_p3(cbase):
            def p3(j, c):
                for u in range(U):
                    off = j * (L * U) + u * L
                    pbuf[pl.ds(off, L)] = smrow[pl.ds(cbase + off, L)] * vscale
                return c
            return p3

        for i in range(NCH):
            lax.fori_loop(0, CH // (L * U), make_p3(i * CH), 0)
            pltpu.sync_copy(pbuf, probs_hbm.at[pl.ds(base + i * CH, CH)])

        va = plsc.load_gather(actbuf, [zerosi + b])
        e_a = plsc.load_gather(smrow, [va])
        vpa = vscale * e_a
        auxv = jnp.where(lanes == 0, vpa,
               jnp.where(lanes == 1, E,
               jnp.where(lanes == 2, EZ,
               jnp.where(lanes == 3, vscale, np.float32(0.0)))))
        auxbuf[...] = auxv
        pltpu.sync_copy(auxbuf, aux_hbm.at[b])
        return 0

    lax.fori_loop(0, RPW, row_body, 0)


_sc_call = functools.partial(
    pl.kernel,
    out_type=(
        jax.ShapeDtypeStruct((B * N,), jnp.float32),
        jax.ShapeDtypeStruct((B, L), jnp.float32),
    ),
    mesh=plsc.VectorSubcoreMesh(core_axis_name="c", subcore_axis_name="s"),
    compiler_params=pltpu.CompilerParams(needs_layout_passes=False),
    scratch_types=[
        pltpu.VMEM((N,), jnp.float32),
        pltpu.VMEM((CH,), jnp.float32),
        pltpu.VMEM((CH,), jnp.int32),
        pltpu.VMEM((CH,), jnp.float32),
        pltpu.VMEM((B,), jnp.int32),
        pltpu.VMEM((L,), jnp.float32),
        pltpu.VMEM((128,), jnp.float32),
    ],
)(_sc_body)


def kernel(scores, available, action):
    probs_flat, aux = _sc_call(
        scores.reshape(-1), available.reshape(-1), action)
    probs = probs_flat.reshape(B, N)
    pa = aux[:, 0]
    E = aux[:, 1]
    EZ = aux[:, 2]
    scale = aux[:, 3]
    action_logprobs = jnp.log(jnp.maximum(pa, 1e-30))
    ls = jnp.log(jnp.maximum(scale, 1e-30))
    dist_entropy = -(scale * EZ + ls * scale * E)
    return action_logprobs, dist_entropy, probs


# speedup vs baseline: 1.3188x; 1.3188x over previous
import functools

import jax
import jax.numpy as jnp
import numpy as np
from jax import lax
from jax.experimental import pallas as pl
from jax.experimental.pallas import tpu as pltpu
from jax.experimental.pallas import tpu_sc as plsc

B = 128
N = 100000
NC, NS, L = 2, 16, 16
NW = NC * NS
RPW = B // NW
CH = 4000
NCH = N // CH
U = 5
BIG = np.float32(3.0e38)
SHIFT = np.float32(12.0)


def _xlane(redbuf, lanes, v, op):
    cur = v
    for s in (8, 4, 2, 1):
        redbuf[pl.ds(0, L)] = cur
        idx = jnp.bitwise_and(lanes + s, L - 1)
        cur = op(cur, plsc.load_gather(redbuf, [idx]))
    return cur


def _sc_body(scores_hbm, avail_hbm, action_hbm, probs_hbm, aux_hbm,
             smrow, sbuf0, sbuf1, abuf0, abuf1, pbuf0, pbuf1,
             actbuf, auxbuf, redbuf, isem0, isem1, osem0, osem1):
    w = lax.axis_index("s") * NC + lax.axis_index("c")
    pltpu.sync_copy(action_hbm, actbuf)
    lanes = lax.iota(jnp.int32, L)
    zeros16 = jnp.zeros((L,), jnp.float32)
    zerosi = jnp.zeros((L,), jnp.int32)
    sbufs = (sbuf0, sbuf1)
    abufs = (abuf0, abuf1)
    pbufs = (pbuf0, pbuf1)
    isems = (isem0, isem1)
    osems = (osem0, osem1)

    def row_body(k, _):
        b = w * RPW + k
        base = b * N

        def start_in(i, par):
            hs = pltpu.async_copy(
                scores_hbm.at[pl.ds(base + i * CH, CH)], sbufs[par], isems[par])
            ha = pltpu.async_copy(
                avail_hbm.at[pl.ds(base + i * CH, CH)], abufs[par], isems[par])
            return hs, ha

        def make_pa(cbase, par):
            sb, ab = sbufs[par], abufs[par]

            def pa(j, c):
                vm, ve, vez, vni = c
                for u in range(U):
                    off = j * (L * U) + u * L
                    s = sb[pl.ds(off, L)]
                    a = ab[pl.ds(off, L)]
                    t = jnp.where(a > 0, s, -BIG) - SHIFT
                    e = jnp.exp(t)
                    smrow[pl.ds(cbase + off, L)] = e
                    vm = jnp.maximum(vm, t)
                    ve = ve + e
                    vez = vez + e * s
                    vni = vni + a
                return vm, ve, vez, vni

            return pa

        carry = (jnp.full((L,), -BIG, jnp.float32), zeros16, zeros16, zerosi)
        h = start_in(0, 0)
        for i in range(NCH):
            par = i & 1
            nh = start_in(i + 1, 1 - par) if i + 1 < NCH else None
            h[0].wait()
            h[1].wait()
            carry = lax.fori_loop(0, CH // (L * U), make_pa(i * CH, par), carry)
            h = nh
        vmax, vE, vEZ, vni = carry

        Msh = _xlane(redbuf, lanes, vmax, jnp.maximum)
        Ep = _xlane(redbuf, lanes, vE, jnp.add)
        EZp = _xlane(redbuf, lanes, vEZ, jnp.add)
        nav = _xlane(redbuf, lanes, vni.astype(jnp.float32), jnp.add)

        has = nav > np.float32(0.0)
        Msh = jnp.where(has, Msh, np.float32(0.0))
        corr = jnp.exp(-Msh)
        E = Ep * corr
        EZ = corr * (EZp - (Msh + SHIFT) * Ep)
        vdenom = E + (np.float32(N) - nav)
        vS = E / vdenom
        vscale = (zeros16 + np.float32(1.0)) / (vdenom * (vS + np.float32(1e-13)))
        vscale = jnp.where(has, vscale, np.float32(0.0))
        tot = corr * vscale

        def make_pb(cbase, par):
            pb = pbufs[par]

            def pbody(j, c):
                for u in range(U):
                    off = j * (L * U) + u * L
                    pb[pl.ds(off, L)] = smrow[pl.ds(cbase + off, L)] * tot
                return c

            return pbody

        oh = [None, None]
        for i in range(NCH):
            par = i & 1
            if oh[par] is not None:
                oh[par].wait()
            lax.fori_loop(0, CH // (L * U), make_pb(i * CH, par), 0)
            oh[par] = pltpu.async_copy(
                pbufs[par], probs_hbm.at[pl.ds(base + i * CH, CH)], osems[par])
        oh[0].wait()
        oh[1].wait()

        va = plsc.load_gather(actbuf, [zerosi + b])
        e_a = plsc.load_gather(smrow, [va])
        vpa = tot * e_a
        auxv = jnp.where(lanes == 0, vpa,
               jnp.where(lanes == 1, E,
               jnp.where(lanes == 2, EZ,
               jnp.where(lanes == 3, vscale, np.float32(0.0)))))
        auxbuf[...] = auxv
        pltpu.sync_copy(auxbuf, aux_hbm.at[b])
        return 0

    lax.fori_loop(0, RPW, row_body, 0)


_sc_call = functools.partial(
    pl.kernel,
    out_type=(
        jax.ShapeDtypeStruct((B * N,), jnp.float32),
        jax.ShapeDtypeStruct((B, L), jnp.float32),
    ),
    mesh=plsc.VectorSubcoreMesh(core_axis_name="c", subcore_axis_name="s"),
    compiler_params=pltpu.CompilerParams(needs_layout_passes=False),
    scratch_types=[
        pltpu.VMEM((N,), jnp.float32),
        pltpu.VMEM((CH,), jnp.float32),
        pltpu.VMEM((CH,), jnp.float32),
        pltpu.VMEM((CH,), jnp.int32),
        pltpu.VMEM((CH,), jnp.int32),
        pltpu.VMEM((CH,), jnp.float32),
        pltpu.VMEM((CH,), jnp.float32),
        pltpu.VMEM((B,), jnp.int32),
        pltpu.VMEM((L,), jnp.float32),
        pltpu.VMEM((128,), jnp.float32),
        pltpu.SemaphoreType.DMA,
        pltpu.SemaphoreType.DMA,
        pltpu.SemaphoreType.DMA,
        pltpu.SemaphoreType.DMA,
    ],
)(_sc_body)


def kernel(scores, available, action):
    probs_flat, aux = _sc_call(
        scores.reshape(-1), available.reshape(-1), action)
    probs = probs_flat.reshape(B, N)
    pa = aux[:, 0]
    E = aux[:, 1]
    EZ = aux[:, 2]
    scale = aux[:, 3]
    action_logprobs = jnp.log(jnp.maximum(pa, 1e-30))
    ls = jnp.log(jnp.maximum(scale, 1e-30))
    dist_entropy = -(scale * EZ + ls * scale * E)
    return action_logprobs, dist_entropy, probs
